# baseline (speedup 1.0000x reference)
import functools
import math

import jax
import jax.numpy as jnp
from jax.experimental import pallas as pl
from jax.experimental.pallas import tpu as pltpu

_S, _H = 2048, 1024
_NH, _NL, _MS, _TK = 16, 64, 512, 32
_HD = _H // _NH
_BLK = 256
_NBLK = _S // _BLK
_NEG = float("-inf")


def _body(hs3_ref, hs_ref, wq_ref, bq_ref, wk_ref, bk_ref, wv_ref, bv_ref,
          wo_ref, bo_ref, mk_ref, mv_ref, wg_ref, bg_ref, out_ref,
          p_ref, k_ref, v_ref, wqb_ref, wob_ref, mkb_ref, mvb_ref):
    i = pl.program_id(0)
    f32, bf16 = jnp.float32, jnp.bfloat16

    @pl.when(i == 0)
    def _prologue():
        wqb_ref[...] = wq_ref[...].astype(bf16)
        wob_ref[...] = wo_ref[...].astype(bf16)
        mkb_ref[...] = mk_ref[...].astype(bf16)
        mvb_ref[...] = mv_ref[...].astype(bf16)
        hs3 = hs3_ref[...]
        imp = jnp.sqrt(jnp.sum(hs3 * hs3, axis=2))
        work = imp
        for j in range(_NL):
            m = jnp.max(work, axis=(0, 1), keepdims=True)
            hit = work >= m
            p_ref[j, :, :] = hit.astype(bf16)
            work = jnp.where(hit, _NEG, work)
        lm = jnp.zeros((_NL, _H), dtype=f32)
        for r in range(16):
            lm = lm + jnp.dot(p_ref[:, r, :], hs3[r].astype(bf16),
                              preferred_element_type=f32)
        k_ref[...] = jnp.dot(lm, wk_ref[...].T,
                             preferred_element_type=f32) + bk_ref[...]
        v_ref[...] = jnp.dot(lm, wv_ref[...].T,
                             preferred_element_type=f32) + bv_ref[...]

    @pl.when(i > 0)
    def _block():
        hs = hs_ref[...]
        hsb = hs.astype(bf16)
        q = jnp.dot(hsb, wqb_ref[...].T,
                    preferred_element_type=f32) + bq_ref[...]
        k = k_ref[...].astype(bf16)
        v = v_ref[...].astype(bf16)
        scale = 1.0 / math.sqrt(_HD)
        ctx_parts = []
        for h in range(_NH):
            sl = slice(h * _HD, (h + 1) * _HD)
            qh = q[:, sl].astype(bf16)
            s = jnp.dot(qh, k[:, sl].T, preferred_element_type=f32) * scale
            s = s - jnp.max(s, axis=1, keepdims=True)
            e = jnp.exp(s)
            a = (e / jnp.sum(e, axis=1, keepdims=True)).astype(bf16)
            ctx_parts.append(jnp.dot(a, v[:, sl], preferred_element_type=f32))
        ctx = jnp.concatenate(ctx_parts, axis=1).astype(bf16)
        att = jnp.dot(ctx, wob_ref[...].T,
                      preferred_element_type=f32) + bo_ref[...]
        attb = att.astype(bf16)
        ms = jnp.dot(attb, mkb_ref[...].T,
                     preferred_element_type=f32) * (1.0 / math.sqrt(_H))
        work = ms
        for _ in range(_TK):
            m = jnp.max(work, axis=1, keepdims=True)
            work = jnp.where(work >= m, _NEG, work)
        sel = work == _NEG
        masked = jnp.where(sel, ms, _NEG)
        mx = jnp.max(masked, axis=1, keepdims=True)
        e = jnp.exp(masked - mx)
        w = (e / jnp.sum(e, axis=1, keepdims=True)).astype(bf16)
        mo = jnp.dot(w, mvb_ref[...], preferred_element_type=f32)
        gate = jax.nn.sigmoid(
            jnp.sum(att * wg_ref[...], axis=1, keepdims=True) + bg_ref[0, 0])
        out_ref[...] = hs + att + gate * mo


def kernel(hidden_states, Wq, bq, Wk, bk, Wv, bv, Wo, bo,
           mem_keys, mem_values, Wg, bg):
    f32, bf16 = jnp.float32, jnp.bfloat16
    hs = hidden_states.reshape(_S, _H)
    hs3 = hidden_states.reshape(16, 128, _H)
    bq2 = bq.reshape(1, _H)
    bk2 = bk.reshape(1, _H)
    bv2 = bv.reshape(1, _H)
    bo2 = bo.reshape(1, _H)
    wg2 = Wg.reshape(1, _H)
    bg2 = bg.reshape(1, 1)

    res = lambda shape: pl.BlockSpec(shape, lambda i: tuple(0 for _ in shape))
    blk_map = lambda i: (jnp.maximum(i - 1, 0), 0)
    out = pl.pallas_call(
        _body,
        grid=(_NBLK + 1,),
        in_specs=[
            res((16, 128, _H)),
            pl.BlockSpec((_BLK, _H), blk_map),
            res((_H, _H)),
            res((1, _H)),
            res((_H, _H)),
            res((1, _H)),
            res((_H, _H)),
            res((1, _H)),
            res((_H, _H)),
            res((1, _H)),
            res((_MS, _H)),
            res((_MS, _H)),
            res((1, _H)),
            res((1, 1)),
        ],
        out_specs=pl.BlockSpec((_BLK, _H), blk_map),
        out_shape=jax.ShapeDtypeStruct((_S, _H), f32),
        scratch_shapes=[
            pltpu.VMEM((_NL, 16, 128), bf16),
            pltpu.VMEM((_NL, _H), f32),
            pltpu.VMEM((_NL, _H), f32),
            pltpu.VMEM((_H, _H), bf16),
            pltpu.VMEM((_H, _H), bf16),
            pltpu.VMEM((_MS, _H), bf16),
            pltpu.VMEM((_MS, _H), bf16),
        ],
    )(hs3, hs, Wq, bq2, Wk, bk2, Wv, bv2, Wo, bo2,
      mem_keys, mem_values, wg2, bg2)
    return out.reshape(1, _S, _H)


# speedup vs baseline: 1.1603x; 1.1603x over previous
import functools
import math

import jax
import jax.numpy as jnp
from jax import lax
from jax.experimental import pallas as pl
from jax.experimental.pallas import tpu as pltpu
from jax.experimental.pallas import tpu_sc as plsc

_S, _H = 2048, 1024
_NH, _NL, _MS, _TK = 16, 64, 512, 32
_HD = _H // _NH
_BLK = 256
_NBLK = _S // _BLK
_NEG = float("-inf")
_RPW = _S // 32


def _landmark_kv_body(hs3_ref, wk_ref, bk_ref, wv_ref, bv_ref,
                      k_ref, v_ref, p_ref):
    hs3 = hs3_ref[...]
    imp = jnp.sqrt(jnp.sum(hs3 * hs3, axis=2))
    r_io = lax.broadcasted_iota(jnp.int32, (16, 128), 0)
    c_io = lax.broadcasted_iota(jnp.int32, (16, 128), 1)
    flat = r_io * 128 + c_io
    work = imp
    for i in range(_NL):
        m = jnp.max(work, axis=(0, 1), keepdims=True)
        idx = jnp.min(jnp.where(work >= m, flat, _S),
                      axis=(0, 1), keepdims=True)
        hit = flat == idx
        p_ref[i, :, :] = hit.astype(jnp.bfloat16)
        work = jnp.where(hit, _NEG, work)
    hs3b = hs3.astype(jnp.bfloat16)
    lm = jnp.zeros((_NL, _H), dtype=jnp.float32)
    for r in range(16):
        lm = lm + jnp.dot(p_ref[:, r, :], hs3b[r],
                          preferred_element_type=jnp.float32)
    lmb = lm.astype(jnp.bfloat16)
    k_ref[...] = jnp.dot(lmb, wk_ref[...].T,
                         preferred_element_type=jnp.float32) + bk_ref[...]
    v_ref[...] = jnp.dot(lmb, wv_ref[...].T,
                         preferred_element_type=jnp.float32) + bv_ref[...]


def _attn_ms_body(hs_ref, wq_ref, bq_ref, wo_ref, bo_ref, mk_ref,
                  k_ref, v_ref, att_ref, ms_ref):
    hs = hs_ref[...]
    hsb = hs.astype(jnp.bfloat16)
    q = jnp.dot(hsb, wq_ref[...].T,
                preferred_element_type=jnp.float32) + bq_ref[...]
    k = k_ref[...].astype(jnp.bfloat16)
    v = v_ref[...].astype(jnp.bfloat16)
    scale = 1.0 / math.sqrt(_HD)
    ctx_parts = []
    for h in range(_NH):
        sl = slice(h * _HD, (h + 1) * _HD)
        qh = q[:, sl].astype(jnp.bfloat16)
        s = jnp.dot(qh, k[:, sl].T, preferred_element_type=jnp.float32) * scale
        s = s - jnp.max(s, axis=1, keepdims=True)
        e = jnp.exp(s)
        a = (e / jnp.sum(e, axis=1, keepdims=True)).astype(jnp.bfloat16)
        ctx_parts.append(jnp.dot(a, v[:, sl], preferred_element_type=jnp.float32))
    ctx = jnp.concatenate(ctx_parts, axis=1).astype(jnp.bfloat16)
    att = jnp.dot(ctx, wo_ref[...].T,
                  preferred_element_type=jnp.float32) + bo_ref[...]
    att_ref[...] = att
    ms_ref[...] = jnp.dot(att.astype(jnp.bfloat16), mk_ref[...].T,
                          preferred_element_type=jnp.float32) * (1.0 / math.sqrt(_H))


def _sorta(x):
    r = plsc.sort_key_val(x, x)
    return r[0] if isinstance(r, (tuple, list)) else r


def _sc_topk_body(ms_hbm, th_hbm, msv, thv):
    cid = lax.axis_index("c")
    sid = lax.axis_index("s")
    wid = sid * 2 + cid
    base = wid * _RPW
    pltpu.sync_copy(ms_hbm.at[pl.ds(base, _RPW), :], msv)
    lane = lax.iota(jnp.int32, 16)

    def row_body(r, accs):
        xs = [_sorta(msv[r, pl.ds(16 * j, 16)]) for j in range(32)]
        runs = []
        for p in range(16):
            a, b = xs[2 * p], xs[2 * p + 1]
            br = lax.rev(b, (0,))
            runs.append((_sorta(jnp.minimum(a, br)),
                         _sorta(jnp.maximum(a, br))))
        while len(runs) > 1:
            nxt = []
            for p in range(len(runs) // 2):
                (a0, a1), (b0, b1) = runs[2 * p], runs[2 * p + 1]
                c0 = jnp.maximum(a0, lax.rev(b1, (0,)))
                c1 = jnp.maximum(a1, lax.rev(b0, (0,)))
                d0 = jnp.minimum(c0, c1)
                d1 = jnp.maximum(c0, c1)
                nxt.append((_sorta(d0), _sorta(d1)))
            runs = nxt
        lo, _ = runs[0]
        th = jnp.min(lo)
        g = r // 16
        l = jnp.remainder(r, 16)
        return tuple(
            jnp.where(jnp.logical_and(g == gi, lane == l), th, accs[gi])
            for gi in range(_RPW // 16))

    zero = jnp.zeros((16,), jnp.float32)
    accs = lax.fori_loop(0, _RPW, row_body,
                         tuple(zero for _ in range(_RPW // 16)))
    for gi in range(_RPW // 16):
        thv[pl.ds(16 * gi, 16)] = accs[gi]
    pltpu.sync_copy(thv, th_hbm.at[pl.ds(base, _RPW)])


def _finish_body(hs_ref, att_ref, ms_ref, th_ref, mv_ref, wg_ref, bg_ref,
                 out_ref):
    hs = hs_ref[...]
    att = att_ref[...]
    ms = ms_ref[...]
    thc = th_ref[0]
    sel = ms >= thc
    masked = jnp.where(sel, ms, _NEG)
    mx = jnp.max(masked, axis=1, keepdims=True)
    e = jnp.exp(masked - mx)
    w = (e / jnp.sum(e, axis=1, keepdims=True)).astype(jnp.bfloat16)
    mo = jnp.dot(w, mv_ref[...], preferred_element_type=jnp.float32)
    gate = jax.nn.sigmoid(
        jnp.sum(att * wg_ref[...], axis=1, keepdims=True) + bg_ref[0, 0])
    out_ref[...] = hs + att + gate * mo


def kernel(hidden_states, Wq, bq, Wk, bk, Wv, bv, Wo, bo,
           mem_keys, mem_values, Wg, bg):
    f32, bf16 = jnp.float32, jnp.bfloat16
    hs = hidden_states.reshape(_S, _H)
    hs3 = hidden_states.reshape(16, 128, _H)
    bk2 = bk.reshape(1, _H)
    bv2 = bv.reshape(1, _H)
    bq2 = bq.reshape(1, _H)
    bo2 = bo.reshape(1, _H)
    wg2 = Wg.reshape(1, _H)
    bg2 = bg.reshape(1, 1)

    k, v = pl.pallas_call(
        _landmark_kv_body,
        out_shape=(
            jax.ShapeDtypeStruct((_NL, _H), f32),
            jax.ShapeDtypeStruct((_NL, _H), f32),
        ),
        scratch_shapes=[pltpu.VMEM((_NL, 16, 128), bf16)],
    )(hs3, Wk.astype(bf16), bk2, Wv.astype(bf16), bv2)

    full = lambda shape: pl.BlockSpec(shape, lambda i: tuple(0 for _ in shape))
    blk = pl.BlockSpec((_BLK, _H), lambda i: (i, 0))
    att, ms = pl.pallas_call(
        _attn_ms_body,
        grid=(_NBLK,),
        in_specs=[
            blk,
            full((_H, _H)),
            full((1, _H)),
            full((_H, _H)),
            full((1, _H)),
            full((_MS, _H)),
            full((_NL, _H)),
            full((_NL, _H)),
        ],
        out_specs=(blk, pl.BlockSpec((_BLK, _MS), lambda i: (i, 0))),
        out_shape=(
            jax.ShapeDtypeStruct((_S, _H), f32),
            jax.ShapeDtypeStruct((_S, _MS), f32),
        ),
    )(hs, Wq.astype(bf16), bq2, Wo.astype(bf16), bo2, mem_keys.astype(bf16),
      k, v)

    mesh = plsc.VectorSubcoreMesh(core_axis_name="c", subcore_axis_name="s")
    th = pl.kernel(
        _sc_topk_body,
        mesh=mesh,
        compiler_params=pltpu.CompilerParams(needs_layout_passes=False),
        out_type=jax.ShapeDtypeStruct((_S,), f32),
        scratch_types=[
            pltpu.VMEM((_RPW, _MS), f32),
            pltpu.VMEM((_RPW,), f32),
        ],
    )(ms)

    th3 = th.reshape(_NBLK, _BLK, 1)
    out = pl.pallas_call(
        _finish_body,
        grid=(_NBLK,),
        in_specs=[
            blk,
            blk,
            pl.BlockSpec((_BLK, _MS), lambda i: (i, 0)),
            pl.BlockSpec((1, _BLK, 1), lambda i: (i, 0, 0)),
            full((_MS, _H)),
            full((1, _H)),
            full((1, 1)),
        ],
        out_specs=blk,
        out_shape=jax.ShapeDtypeStruct((_S, _H), f32),
    )(hs, att, ms, th3, mem_values.astype(bf16), wg2, bg2)
    return out.reshape(1, _S, _H)


# speedup vs baseline: 1.1704x; 1.0087x over previous
import functools
import math

import jax
import jax.numpy as jnp
from jax import lax
from jax.experimental import pallas as pl
from jax.experimental.pallas import tpu as pltpu
from jax.experimental.pallas import tpu_sc as plsc

_S, _H = 2048, 1024
_NH, _NL, _MS, _TK = 16, 64, 512, 32
_HD = _H // _NH
_BLK = 256
_NBLK = _S // _BLK
_NEG = float("-inf")
_RPW = _S // 32


def _landmark_kv_body(hs3_ref, wk_ref, bk_ref, wv_ref, bv_ref,
                      k_ref, v_ref, p_ref):
    hs3 = hs3_ref[...]
    imp = jnp.sqrt(jnp.sum(hs3 * hs3, axis=2))
    r_io = lax.broadcasted_iota(jnp.int32, (16, 128), 0)
    c_io = lax.broadcasted_iota(jnp.int32, (16, 128), 1)
    flat = r_io * 128 + c_io
    work = imp
    for i in range(_NL):
        m = jnp.max(work, axis=(0, 1), keepdims=True)
        idx = jnp.min(jnp.where(work >= m, flat, _S),
                      axis=(0, 1), keepdims=True)
        hit = flat == idx
        p_ref[i, :, :] = hit.astype(jnp.bfloat16)
        work = jnp.where(hit, _NEG, work)
    hs3b = hs3.astype(jnp.bfloat16)
    lm = jnp.zeros((_NL, _H), dtype=jnp.float32)
    for r in range(16):
        lm = lm + jnp.dot(p_ref[:, r, :], hs3b[r],
                          preferred_element_type=jnp.float32)
    lmb = lm.astype(jnp.bfloat16)
    k_ref[...] = jnp.dot(lmb, wk_ref[...].T,
                         preferred_element_type=jnp.float32) + bk_ref[...]
    v_ref[...] = jnp.dot(lmb, wv_ref[...].T,
                         preferred_element_type=jnp.float32) + bv_ref[...]


def _attn_ms_body(hs_ref, wq_ref, bq_ref, wo_ref, bo_ref, mk_ref,
                  wg_ref, bg_ref, k_ref, v_ref, part_ref, gate_ref, ms_ref):
    hs = hs_ref[...]
    hsb = hs.astype(jnp.bfloat16)
    q = jnp.dot(hsb, wq_ref[...].T,
                preferred_element_type=jnp.float32) + bq_ref[...]
    k = k_ref[...].astype(jnp.bfloat16)
    v = v_ref[...].astype(jnp.bfloat16)
    scale = 1.0 / math.sqrt(_HD)
    ctx_parts = []
    for h in range(_NH):
        sl = slice(h * _HD, (h + 1) * _HD)
        qh = q[:, sl].astype(jnp.bfloat16)
        s = jnp.dot(qh, k[:, sl].T, preferred_element_type=jnp.float32) * scale
        s = s - jnp.max(s, axis=1, keepdims=True)
        e = jnp.exp(s)
        a = (e / jnp.sum(e, axis=1, keepdims=True)).astype(jnp.bfloat16)
        ctx_parts.append(jnp.dot(a, v[:, sl], preferred_element_type=jnp.float32))
    ctx = jnp.concatenate(ctx_parts, axis=1).astype(jnp.bfloat16)
    att = jnp.dot(ctx, wo_ref[...].T,
                  preferred_element_type=jnp.float32) + bo_ref[...]
    part_ref[...] = hs + att
    gate_ref[...] = jax.nn.sigmoid(
        jnp.sum(att * wg_ref[...], axis=1, keepdims=True) + bg_ref[0, 0])
    ms_ref[...] = jnp.dot(att.astype(jnp.bfloat16), mk_ref[...].T,
                          preferred_element_type=jnp.float32) * (1.0 / math.sqrt(_H))


def _sorta(x):
    r = plsc.sort_key_val(x, x)
    return r[0] if isinstance(r, (tuple, list)) else r


def _sc_topk_body(ms_hbm, th_hbm, msv, thv):
    cid = lax.axis_index("c")
    sid = lax.axis_index("s")
    wid = sid * 2 + cid
    base = wid * _RPW
    pltpu.sync_copy(ms_hbm.at[pl.ds(base, _RPW), :], msv)
    lane = lax.iota(jnp.int32, 16)

    def row_body(r, accs):
        xs = [_sorta(msv[r, pl.ds(16 * j, 16)]) for j in range(32)]
        runs = []
        for p in range(16):
            a, b = xs[2 * p], xs[2 * p + 1]
            br = lax.rev(b, (0,))
            runs.append((_sorta(jnp.minimum(a, br)),
                         _sorta(jnp.maximum(a, br))))
        while len(runs) > 1:
            nxt = []
            for p in range(len(runs) // 2):
                (a0, a1), (b0, b1) = runs[2 * p], runs[2 * p + 1]
                c0 = jnp.maximum(a0, lax.rev(b1, (0,)))
                c1 = jnp.maximum(a1, lax.rev(b0, (0,)))
                d0 = jnp.minimum(c0, c1)
                d1 = jnp.maximum(c0, c1)
                nxt.append((_sorta(d0), _sorta(d1)))
            runs = nxt
        lo, _ = runs[0]
        th = jnp.min(lo)
        g = r // 16
        l = jnp.remainder(r, 16)
        return tuple(
            jnp.where(jnp.logical_and(g == gi, lane == l), th, accs[gi])
            for gi in range(_RPW // 16))

    zero = jnp.zeros((16,), jnp.float32)
    accs = lax.fori_loop(0, _RPW, row_body,
                         tuple(zero for _ in range(_RPW // 16)))
    for gi in range(_RPW // 16):
        thv[pl.ds(16 * gi, 16)] = accs[gi]
    pltpu.sync_copy(thv, th_hbm.at[pl.ds(base, _RPW)])


def _finish_body(part_ref, gate_ref, ms_ref, th_ref, mv_ref, out_ref):
    ms = ms_ref[...]
    thc = th_ref[0]
    sel = ms >= thc
    masked = jnp.where(sel, ms, _NEG)
    mx = jnp.max(masked, axis=1, keepdims=True)
    e = jnp.exp(masked - mx)
    w = (e / jnp.sum(e, axis=1, keepdims=True)).astype(jnp.bfloat16)
    mo = jnp.dot(w, mv_ref[...], preferred_element_type=jnp.float32)
    out_ref[...] = part_ref[...] + gate_ref[...] * mo


def kernel(hidden_states, Wq, bq, Wk, bk, Wv, bv, Wo, bo,
           mem_keys, mem_values, Wg, bg):
    f32, bf16 = jnp.float32, jnp.bfloat16
    hs = hidden_states.reshape(_S, _H)
    hs3 = hidden_states.reshape(16, 128, _H)
    bk2 = bk.reshape(1, _H)
    bv2 = bv.reshape(1, _H)
    bq2 = bq.reshape(1, _H)
    bo2 = bo.reshape(1, _H)
    wg2 = Wg.reshape(1, _H)
    bg2 = bg.reshape(1, 1)

    k, v = pl.pallas_call(
        _landmark_kv_body,
        out_shape=(
            jax.ShapeDtypeStruct((_NL, _H), f32),
            jax.ShapeDtypeStruct((_NL, _H), f32),
        ),
        scratch_shapes=[pltpu.VMEM((_NL, 16, 128), bf16)],
    )(hs3, Wk.astype(bf16), bk2, Wv.astype(bf16), bv2)

    full = lambda shape: pl.BlockSpec(shape, lambda i: tuple(0 for _ in shape))
    blk = pl.BlockSpec((_BLK, _H), lambda i: (i, 0))
    part, gate, ms = pl.pallas_call(
        _attn_ms_body,
        grid=(_NBLK,),
        in_specs=[
            blk,
            full((_H, _H)),
            full((1, _H)),
            full((_H, _H)),
            full((1, _H)),
            full((_MS, _H)),
            full((1, _H)),
            full((1, 1)),
            full((_NL, _H)),
            full((_NL, _H)),
        ],
        out_specs=(blk,
                   pl.BlockSpec((_BLK, 1), lambda i: (i, 0)),
                   pl.BlockSpec((_BLK, _MS), lambda i: (i, 0))),
        out_shape=(
            jax.ShapeDtypeStruct((_S, _H), f32),
            jax.ShapeDtypeStruct((_S, 1), f32),
            jax.ShapeDtypeStruct((_S, _MS), f32),
        ),
    )(hs, Wq.astype(bf16), bq2, Wo.astype(bf16), bo2, mem_keys.astype(bf16),
      wg2, bg2, k, v)

    mesh = plsc.VectorSubcoreMesh(core_axis_name="c", subcore_axis_name="s")
    th = pl.kernel(
        _sc_topk_body,
        mesh=mesh,
        compiler_params=pltpu.CompilerParams(needs_layout_passes=False),
        out_type=jax.ShapeDtypeStruct((_S,), f32),
        scratch_types=[
            pltpu.VMEM((_RPW, _MS), f32),
            pltpu.VMEM((_RPW,), f32),
        ],
    )(ms)

    th3 = th.reshape(_NBLK, _BLK, 1)
    out = pl.pallas_call(
        _finish_body,
        grid=(_NBLK,),
        in_specs=[
            blk,
            pl.BlockSpec((_BLK, 1), lambda i: (i, 0)),
            pl.BlockSpec((_BLK, _MS), lambda i: (i, 0)),
            pl.BlockSpec((1, _BLK, 1), lambda i: (i, 0, 0)),
            full((_MS, _H)),
        ],
        out_specs=blk,
        out_shape=jax.ShapeDtypeStruct((_S, _H), f32),
    )(part, gate, ms, th3, mem_values.astype(bf16))
    return out.reshape(1, _S, _H)


# speedup vs baseline: 1.2086x; 1.0327x over previous
import functools
import math

import jax
import jax.numpy as jnp
from jax import lax
from jax.experimental import pallas as pl
from jax.experimental.pallas import tpu as pltpu
from jax.experimental.pallas import tpu_sc as plsc

_S, _H = 2048, 1024
_NH, _NL, _MS, _TK = 16, 64, 512, 32
_HD = _H // _NH
_BLK = 512
_NBLK = _S // _BLK
_NEG = float("-inf")
_RPW = _S // 32


def _landmark_kv_body(hs3_ref, wk_ref, bk_ref, wv_ref, bv_ref,
                      k_ref, v_ref, p_ref):
    hs3 = hs3_ref[...]
    imp = jnp.sqrt(jnp.sum(hs3 * hs3, axis=2))
    r_io = lax.broadcasted_iota(jnp.int32, (16, 128), 0)
    c_io = lax.broadcasted_iota(jnp.int32, (16, 128), 1)
    flat = r_io * 128 + c_io
    work = imp
    for i in range(_NL):
        m = jnp.max(work, axis=(0, 1), keepdims=True)
        idx = jnp.min(jnp.where(work >= m, flat, _S),
                      axis=(0, 1), keepdims=True)
        hit = flat == idx
        p_ref[i, :, :] = hit.astype(jnp.bfloat16)
        work = jnp.where(hit, _NEG, work)
    hs3b = hs3.astype(jnp.bfloat16)
    lm = jnp.zeros((_NL, _H), dtype=jnp.float32)
    for r in range(16):
        lm = lm + jnp.dot(p_ref[:, r, :], hs3b[r],
                          preferred_element_type=jnp.float32)
    lmb = lm.astype(jnp.bfloat16)
    k_ref[...] = jnp.dot(lmb, wk_ref[...].T,
                         preferred_element_type=jnp.float32) + bk_ref[...]
    v_ref[...] = jnp.dot(lmb, wv_ref[...].T,
                         preferred_element_type=jnp.float32) + bv_ref[...]


def _attn_ms_body(hs_ref, wq_ref, bq_ref, wo_ref, bo_ref, mk_ref,
                  wg_ref, bg_ref, k_ref, v_ref, part_ref, gate_ref, ms_ref):
    hs = hs_ref[...]
    hsb = hs.astype(jnp.bfloat16)
    q = jnp.dot(hsb, wq_ref[...].T,
                preferred_element_type=jnp.float32) + bq_ref[...]
    k = k_ref[...].astype(jnp.bfloat16)
    v = v_ref[...].astype(jnp.bfloat16)
    scale = 1.0 / math.sqrt(_HD)
    ctx_parts = []
    for h in range(_NH):
        sl = slice(h * _HD, (h + 1) * _HD)
        qh = q[:, sl].astype(jnp.bfloat16)
        s = jnp.dot(qh, k[:, sl].T, preferred_element_type=jnp.float32) * scale
        s = s - jnp.max(s, axis=1, keepdims=True)
        e = jnp.exp(s)
        a = (e / jnp.sum(e, axis=1, keepdims=True)).astype(jnp.bfloat16)
        ctx_parts.append(jnp.dot(a, v[:, sl], preferred_element_type=jnp.float32))
    ctx = jnp.concatenate(ctx_parts, axis=1).astype(jnp.bfloat16)
    att = jnp.dot(ctx, wo_ref[...].T,
                  preferred_element_type=jnp.float32) + bo_ref[...]
    part_ref[...] = hs + att
    gate_ref[...] = jax.nn.sigmoid(
        jnp.sum(att * wg_ref[...], axis=1, keepdims=True) + bg_ref[0, 0])
    ms_ref[...] = jnp.dot(att.astype(jnp.bfloat16), mk_ref[...].T,
                          preferred_element_type=jnp.float32) * (1.0 / math.sqrt(_H))


def _sorta(x):
    r = plsc.sort_key_val(x, x)
    return r[0] if isinstance(r, (tuple, list)) else r


def _sc_topk_body(ms_hbm, th_hbm, msv, thv):
    cid = lax.axis_index("c")
    sid = lax.axis_index("s")
    wid = sid * 2 + cid
    base = wid * _RPW
    pltpu.sync_copy(ms_hbm.at[pl.ds(base, _RPW), :], msv)
    lane = lax.iota(jnp.int32, 16)

    def row_body(r, accs):
        xs = [_sorta(msv[r, pl.ds(16 * j, 16)]) for j in range(32)]
        runs = []
        for p in range(16):
            a, b = xs[2 * p], xs[2 * p + 1]
            br = lax.rev(b, (0,))
            runs.append((_sorta(jnp.minimum(a, br)),
                         _sorta(jnp.maximum(a, br))))
        while len(runs) > 1:
            nxt = []
            for p in range(len(runs) // 2):
                (a0, a1), (b0, b1) = runs[2 * p], runs[2 * p + 1]
                c0 = jnp.maximum(a0, lax.rev(b1, (0,)))
                c1 = jnp.maximum(a1, lax.rev(b0, (0,)))
                d0 = jnp.minimum(c0, c1)
                d1 = jnp.maximum(c0, c1)
                nxt.append((_sorta(d0), _sorta(d1)))
            runs = nxt
        lo, _ = runs[0]
        th = jnp.min(lo)
        g = r // 16
        l = jnp.remainder(r, 16)
        return tuple(
            jnp.where(jnp.logical_and(g == gi, lane == l), th, accs[gi])
            for gi in range(_RPW // 16))

    zero = jnp.zeros((16,), jnp.float32)
    accs = lax.fori_loop(0, _RPW, row_body,
                         tuple(zero for _ in range(_RPW // 16)))
    for gi in range(_RPW // 16):
        thv[pl.ds(16 * gi, 16)] = accs[gi]
    pltpu.sync_copy(thv, th_hbm.at[pl.ds(base, _RPW)])


def _finish_body(part_ref, gate_ref, ms_ref, th_ref, mv_ref, out_ref):
    ms = ms_ref[...]
    thc = th_ref[0]
    sel = ms >= thc
    masked = jnp.where(sel, ms, _NEG)
    mx = jnp.max(masked, axis=1, keepdims=True)
    e = jnp.exp(masked - mx)
    w = (e / jnp.sum(e, axis=1, keepdims=True)).astype(jnp.bfloat16)
    mo = jnp.dot(w, mv_ref[...], preferred_element_type=jnp.float32)
    out_ref[...] = part_ref[...] + gate_ref[...] * mo


def kernel(hidden_states, Wq, bq, Wk, bk, Wv, bv, Wo, bo,
           mem_keys, mem_values, Wg, bg):
    f32, bf16 = jnp.float32, jnp.bfloat16
    hs = hidden_states.reshape(_S, _H)
    hs3 = hidden_states.reshape(16, 128, _H)
    bk2 = bk.reshape(1, _H)
    bv2 = bv.reshape(1, _H)
    bq2 = bq.reshape(1, _H)
    bo2 = bo.reshape(1, _H)
    wg2 = Wg.reshape(1, _H)
    bg2 = bg.reshape(1, 1)

    k, v = pl.pallas_call(
        _landmark_kv_body,
        out_shape=(
            jax.ShapeDtypeStruct((_NL, _H), f32),
            jax.ShapeDtypeStruct((_NL, _H), f32),
        ),
        scratch_shapes=[pltpu.VMEM((_NL, 16, 128), bf16)],
    )(hs3, Wk.astype(bf16), bk2, Wv.astype(bf16), bv2)

    full = lambda shape: pl.BlockSpec(shape, lambda i: tuple(0 for _ in shape))
    blk = pl.BlockSpec((_BLK, _H), lambda i: (i, 0))
    part, gate, ms = pl.pallas_call(
        _attn_ms_body,
        grid=(_NBLK,),
        in_specs=[
            blk,
            full((_H, _H)),
            full((1, _H)),
            full((_H, _H)),
            full((1, _H)),
            full((_MS, _H)),
            full((1, _H)),
            full((1, 1)),
            full((_NL, _H)),
            full((_NL, _H)),
        ],
        out_specs=(blk,
                   pl.BlockSpec((_BLK, 1), lambda i: (i, 0)),
                   pl.BlockSpec((_BLK, _MS), lambda i: (i, 0))),
        out_shape=(
            jax.ShapeDtypeStruct((_S, _H), f32),
            jax.ShapeDtypeStruct((_S, 1), f32),
            jax.ShapeDtypeStruct((_S, _MS), f32),
        ),
    )(hs, Wq.astype(bf16), bq2, Wo.astype(bf16), bo2, mem_keys.astype(bf16),
      wg2, bg2, k, v)

    mesh = plsc.VectorSubcoreMesh(core_axis_name="c", subcore_axis_name="s")
    th = pl.kernel(
        _sc_topk_body,
        mesh=mesh,
        compiler_params=pltpu.CompilerParams(needs_layout_passes=False),
        out_type=jax.ShapeDtypeStruct((_S,), f32),
        scratch_types=[
            pltpu.VMEM((_RPW, _MS), f32),
            pltpu.VMEM((_RPW,), f32),
        ],
    )(ms)

    th3 = th.reshape(_NBLK, _BLK, 1)
    out = pl.pallas_call(
        _finish_body,
        grid=(_NBLK,),
        in_specs=[
            blk,
            pl.BlockSpec((_BLK, 1), lambda i: (i, 0)),
            pl.BlockSpec((_BLK, _MS), lambda i: (i, 0)),
            pl.BlockSpec((1, _BLK, 1), lambda i: (i, 0, 0)),
            full((_MS, _H)),
        ],
        out_specs=blk,
        out_shape=jax.ShapeDtypeStruct((_S, _H), f32),
    )(part, gate, ms, th3, mem_values.astype(bf16))
    return out.reshape(1, _S, _H)
